# baseline (speedup 1.0000x reference)
import functools

import jax
import jax.numpy as jnp
from jax import lax
from jax.experimental import pallas as pl
from jax.experimental.pallas import tpu as pltpu
from jax.experimental.pallas import tpu_sc as plsc

N = 10000
D_IN = 128
D_OUT = 40

NC, NS, NL = 2, 16, 16
NW = NC * NS
NP = 10240
NSP = 10048
DP = 48
CHUNK = 128
ROWS_PER_TILE = NSP // NS
BN = 2560

_mesh = plsc.VectorSubcoreMesh(core_axis_name="c", subcore_axis_name="s")
_sc_params = pltpu.CompilerParams(use_tc_tiling_on_sc=False)


def _num_chunks(e_pad):
    return e_pad // (NW * CHUNK)


def _make_deg_kernel(e_pad):
    nchunks = _num_chunks(e_pad)
    depth = 4

    @functools.partial(
        pl.kernel,
        out_type=jax.ShapeDtypeStruct((NC * NP, NL), jnp.float32),
        scratch_types=[
            pltpu.VMEM_SHARED((NSP, NL), jnp.float32),
            pltpu.VMEM((CHUNK, NL), jnp.float32),
            pltpu.VMEM((nchunks, CHUNK), jnp.int32),
            pltpu.VMEM((ROWS_PER_TILE, NL), jnp.float32),
            pltpu.SemaphoreType.DMA,
        ],
        mesh=_mesh,
        compiler_params=_sc_params,
    )
    def deg_kernel(dst_hbm, ones_hbm, zeros_hbm, out_hbm,
                   deg_sp, ones_v, dst_slab, zrows_v, sem_s):
        cid = lax.axis_index("c")
        sid = lax.axis_index("s")
        wid = sid * NC + cid
        rbase = sid * ROWS_PER_TILE
        pltpu.sync_copy(ones_hbm, ones_v)
        pltpu.sync_copy(zeros_hbm, zrows_v)
        pltpu.sync_copy(zrows_v, deg_sp.at[pl.ds(rbase, ROWS_PER_TILE)])
        pltpu.sync_copy(dst_hbm.at[wid], dst_slab)
        plsc.subcore_barrier()

        def wait_one():
            pltpu.make_async_copy(zeros_hbm.at[pl.ds(0, CHUNK)], ones_v,
                                  sem_s).wait()

        def body(j, carry):
            @pl.when(j >= depth)
            def _():
                wait_one()
            pltpu.async_copy(ones_v, deg_sp.at[dst_slab.at[j]], sem_s,
                             add=True)
            return carry

        lax.fori_loop(0, nchunks, body, 0)
        for _ in range(depth):
            wait_one()
        plsc.subcore_barrier()
        pltpu.sync_copy(deg_sp.at[pl.ds(rbase, ROWS_PER_TILE)],
                        out_hbm.at[pl.ds(cid * NP + rbase, ROWS_PER_TILE)])

    return deg_kernel


NBUF = 4


def _edge_loop(nchunks, yn_ref, acc_sp, src_slab, dst_slab, rows,
               sem_g, sem_s, dummy_hbm):
    def gather(j, slot):
        pltpu.async_copy(yn_ref.at[src_slab.at[j]], rows.at[slot], sem_g)

    def scatter(j, slot):
        pltpu.async_copy(rows.at[slot], acc_sp.at[dst_slab.at[j]], sem_s,
                         add=True)

    def wait(sem):
        pltpu.make_async_copy(dummy_hbm.at[pl.ds(0, CHUNK)], rows.at[0],
                              sem).wait()

    gather(0, 0)
    gather(1, 1)
    gather(2, 2)

    def body(j, carry):
        wait(sem_g)

        @pl.when(j >= 1)
        def _():
            wait(sem_s)

        @pl.when(j + 3 < nchunks)
        def _():
            gather(j + 3, lax.rem(j + 3, NBUF))

        scatter(j, lax.rem(j, NBUF))
        return carry

    lax.fori_loop(0, nchunks, body, 0)
    wait(sem_s)


def _make_hop_kernel(e_pad):
    nchunks = _num_chunks(e_pad)

    @functools.partial(
        pl.kernel,
        out_type=jax.ShapeDtypeStruct((NC * NP, DP), jnp.float32),
        scratch_types=[
            pltpu.VMEM_SHARED((NSP, DP), jnp.float32),
            pltpu.VMEM((nchunks, CHUNK), jnp.int32),
            pltpu.VMEM((nchunks, CHUNK), jnp.int32),
            pltpu.VMEM((NBUF, CHUNK, DP), jnp.float32),
            pltpu.SemaphoreType.DMA,
            pltpu.SemaphoreType.DMA,
        ],
        mesh=_mesh,
        compiler_params=_sc_params,
    )
    def hop_kernel(yn_hbm, src_hbm, dst_hbm, zeros_hbm, out_hbm,
                   acc_sp, src_slab, dst_slab, rows, sem_g, sem_s):
        cid = lax.axis_index("c")
        sid = lax.axis_index("s")
        wid = sid * NC + cid
        rbase = sid * ROWS_PER_TILE
        pltpu.sync_copy(zeros_hbm, acc_sp.at[pl.ds(rbase, ROWS_PER_TILE)])
        pltpu.sync_copy(src_hbm.at[wid], src_slab)
        pltpu.sync_copy(dst_hbm.at[wid], dst_slab)
        plsc.subcore_barrier()
        _edge_loop(nchunks, yn_hbm, acc_sp, src_slab, dst_slab, rows,
                   sem_g, sem_s, yn_hbm)
        plsc.subcore_barrier()
        pltpu.sync_copy(acc_sp.at[pl.ds(rbase, ROWS_PER_TILE)],
                        out_hbm.at[pl.ds(cid * NP + rbase, ROWS_PER_TILE)])

    return hop_kernel


def _deg_of(degp_ref):
    deg = degp_ref[0, :, 0:1] + degp_ref[1, :, 0:1]
    return jnp.maximum(deg, 1.0)


def _mm_scale_body(degp_ref, x_ref, w_ref, y0_ref):
    y = lax.dot_general(
        x_ref[...], w_ref[...], (((1,), (1,)), ((), ())),
        preferred_element_type=jnp.float32)
    y0_ref[...] = y * lax.rsqrt(_deg_of(degp_ref))


def _scale_mid_body(degp_ref, zp_ref, o_ref):
    z = zp_ref[0] + zp_ref[1]
    o_ref[...] = z / _deg_of(degp_ref)


def _scale_out_body(degp_ref, zp_ref, b_ref, o_ref):
    z = zp_ref[0] + zp_ref[1]
    o_ref[...] = z * lax.rsqrt(_deg_of(degp_ref)) + b_ref[...]


def _deg_spec():
    return pl.BlockSpec((NC, BN, NL), lambda i: (0, i, 0))


def _zp_spec():
    return pl.BlockSpec((NC, BN, DP), lambda i: (0, i, 0))


def _row_spec():
    return pl.BlockSpec((BN, DP), lambda i: (i, 0))


def _mm_scale(degp, xp, wp):
    return pl.pallas_call(
        _mm_scale_body, grid=(NP // BN,),
        in_specs=[_deg_spec(),
                  pl.BlockSpec((BN, D_IN), lambda i: (i, 0)),
                  pl.BlockSpec((DP, D_IN), lambda i: (0, 0))],
        out_specs=_row_spec(),
        out_shape=jax.ShapeDtypeStruct((NP, DP), jnp.float32),
    )(degp, xp, wp)


def _scale_mid(degp, zp):
    return pl.pallas_call(
        _scale_mid_body, grid=(NP // BN,),
        in_specs=[_deg_spec(), _zp_spec()],
        out_specs=_row_spec(),
        out_shape=jax.ShapeDtypeStruct((NP, DP), jnp.float32),
    )(degp, zp)


def _scale_out(degp, zp, bp):
    return pl.pallas_call(
        _scale_out_body, grid=(NP // BN,),
        in_specs=[_deg_spec(), _zp_spec(),
                  pl.BlockSpec((1, DP), lambda i: (0, 0))],
        out_specs=_row_spec(),
        out_shape=jax.ShapeDtypeStruct((NP, DP), jnp.float32),
    )(degp, zp, bp)


def kernel(features, edge_index, W, b):
    E = edge_index.shape[1]
    e_pad = ((E + NW * CHUNK - 1) // (NW * CHUNK)) * (NW * CHUNK)
    deg_kernel = _make_deg_kernel(e_pad)
    hop_kernel = _make_hop_kernel(e_pad)

    src = edge_index[0]
    dst = edge_index[1]
    if e_pad != E:
        pad_idx = jnp.full((e_pad - E,), N, jnp.int32)
        src = jnp.concatenate([src, pad_idx])
        dst = jnp.concatenate([dst, pad_idx])
    nchunks = _num_chunks(e_pad)
    src = src.reshape(NW, nchunks, CHUNK)
    dst = dst.reshape(NW, nchunks, CHUNK)

    xp = jnp.zeros((NP, D_IN), jnp.float32).at[:N].set(features)
    wp = jnp.zeros((DP, D_IN), jnp.float32).at[:D_OUT].set(W)
    bp = jnp.zeros((1, DP), jnp.float32).at[0, :D_OUT].set(b)
    ones_rows = jnp.ones((CHUNK, NL), jnp.float32)
    zeros_deg = jnp.zeros((ROWS_PER_TILE, NL), jnp.float32)
    zeros_hop = jnp.zeros((ROWS_PER_TILE, DP), jnp.float32)

    degp = deg_kernel(dst, ones_rows, zeros_deg)
    degp = degp.reshape(NC, NP, NL)

    y0 = _mm_scale(degp, xp, wp)
    z1 = hop_kernel(y0, src, dst, zeros_hop).reshape(NC, NP, DP)
    y1 = _scale_mid(degp, z1)
    z2 = hop_kernel(y1, src, dst, zeros_hop).reshape(NC, NP, DP)
    out = _scale_out(degp, z2, bp)
    return out[:N, :D_OUT]


# speedup vs baseline: 1.4225x; 1.4225x over previous
import functools

import jax
import jax.numpy as jnp
from jax import lax
from jax.experimental import pallas as pl
from jax.experimental.pallas import tpu as pltpu
from jax.experimental.pallas import tpu_sc as plsc

N = 10000
D_IN = 128
D_OUT = 40

NC, NS, NL = 2, 16, 16
NW = NC * NS
NP = 10240
NSP = 10048
DP = 48
CHUNK = 128
ROWS_PER_TILE = NSP // NS
BN = 2560

_mesh = plsc.VectorSubcoreMesh(core_axis_name="c", subcore_axis_name="s")
_sc_params = pltpu.CompilerParams(use_tc_tiling_on_sc=False)


def _num_chunks(e_pad):
    return e_pad // (NW * CHUNK)


def _make_deg_kernel(e_pad):
    nchunks = _num_chunks(e_pad)
    depth = 4

    @functools.partial(
        pl.kernel,
        out_type=jax.ShapeDtypeStruct((NC * NP, NL), jnp.float32),
        scratch_types=[
            pltpu.VMEM_SHARED((NSP, NL), jnp.float32),
            pltpu.VMEM((CHUNK, NL), jnp.float32),
            pltpu.VMEM((nchunks, CHUNK), jnp.int32),
            pltpu.VMEM((ROWS_PER_TILE, NL), jnp.float32),
            pltpu.SemaphoreType.DMA,
        ],
        mesh=_mesh,
        compiler_params=_sc_params,
    )
    def deg_kernel(dst_hbm, ones_hbm, zeros_hbm, out_hbm,
                   deg_sp, ones_v, dst_slab, zrows_v, sem_s):
        cid = lax.axis_index("c")
        sid = lax.axis_index("s")
        wid = sid * NC + cid
        rbase = sid * ROWS_PER_TILE
        pltpu.sync_copy(ones_hbm, ones_v)
        pltpu.sync_copy(zeros_hbm, zrows_v)
        pltpu.sync_copy(zrows_v, deg_sp.at[pl.ds(rbase, ROWS_PER_TILE)])
        pltpu.sync_copy(dst_hbm.at[wid], dst_slab)
        plsc.subcore_barrier()

        def wait_one():
            pltpu.make_async_copy(zeros_hbm.at[pl.ds(0, CHUNK)], ones_v,
                                  sem_s).wait()

        def body(j, carry):
            @pl.when(j >= depth)
            def _():
                wait_one()
            pltpu.async_copy(ones_v, deg_sp.at[dst_slab.at[j]], sem_s,
                             add=True)
            return carry

        lax.fori_loop(0, nchunks, body, 0)
        for _ in range(depth):
            wait_one()
        plsc.subcore_barrier()
        pltpu.sync_copy(deg_sp.at[pl.ds(rbase, ROWS_PER_TILE)],
                        out_hbm.at[pl.ds(cid * NP + rbase, ROWS_PER_TILE)])

    return deg_kernel


NBUF = 4


def _edge_loop(nchunks, yn_ref, acc_sp, src_slab, dst_slab, rows,
               sem_g, sem_s, dummy_hbm):
    def gather(j, slot):
        pltpu.async_copy(yn_ref.at[src_slab.at[j]], rows.at[slot], sem_g)

    def scatter(j, slot):
        pltpu.async_copy(rows.at[slot], acc_sp.at[dst_slab.at[j]], sem_s,
                         add=True)

    def wait(sem):
        pltpu.make_async_copy(dummy_hbm.at[pl.ds(0, CHUNK)], rows.at[0],
                              sem).wait()

    gather(0, 0)
    gather(1, 1)
    gather(2, 2)

    def body(j, carry):
        wait(sem_g)

        @pl.when(j >= 1)
        def _():
            wait(sem_s)

        @pl.when(j + 3 < nchunks)
        def _():
            gather(j + 3, lax.rem(j + 3, NBUF))

        scatter(j, lax.rem(j, NBUF))
        return carry

    lax.fori_loop(0, nchunks, body, 0)
    wait(sem_s)


def _make_hop_kernel(e_pad):
    nchunks = _num_chunks(e_pad)

    @functools.partial(
        pl.kernel,
        out_type=jax.ShapeDtypeStruct((NC * NP, DP), jnp.float32),
        scratch_types=[
            pltpu.VMEM_SHARED((NSP, DP), jnp.float32),
            pltpu.VMEM_SHARED((NSP, DP), jnp.float32),
            pltpu.VMEM((nchunks, CHUNK), jnp.int32),
            pltpu.VMEM((nchunks, CHUNK), jnp.int32),
            pltpu.VMEM((NBUF, CHUNK, DP), jnp.float32),
            pltpu.SemaphoreType.DMA,
            pltpu.SemaphoreType.DMA,
        ],
        mesh=_mesh,
        compiler_params=_sc_params,
    )
    def hop_kernel(yn_hbm, src_hbm, dst_hbm, zeros_hbm, out_hbm,
                   yn_sp, acc_sp, src_slab, dst_slab, rows, sem_g, sem_s):
        cid = lax.axis_index("c")
        sid = lax.axis_index("s")
        wid = sid * NC + cid
        rbase = sid * ROWS_PER_TILE
        pltpu.sync_copy(yn_hbm.at[pl.ds(rbase, ROWS_PER_TILE)],
                        yn_sp.at[pl.ds(rbase, ROWS_PER_TILE)])
        pltpu.sync_copy(zeros_hbm, acc_sp.at[pl.ds(rbase, ROWS_PER_TILE)])
        pltpu.sync_copy(src_hbm.at[wid], src_slab)
        pltpu.sync_copy(dst_hbm.at[wid], dst_slab)
        plsc.subcore_barrier()
        _edge_loop(nchunks, yn_sp, acc_sp, src_slab, dst_slab, rows,
                   sem_g, sem_s, yn_hbm)
        plsc.subcore_barrier()
        pltpu.sync_copy(acc_sp.at[pl.ds(rbase, ROWS_PER_TILE)],
                        out_hbm.at[pl.ds(cid * NP + rbase, ROWS_PER_TILE)])

    return hop_kernel


def _deg_of(degp_ref):
    deg = degp_ref[0, :, 0:1] + degp_ref[1, :, 0:1]
    return jnp.maximum(deg, 1.0)


def _mm_scale_body(degp_ref, x_ref, w_ref, y0_ref):
    y = lax.dot_general(
        x_ref[...], w_ref[...], (((1,), (1,)), ((), ())),
        preferred_element_type=jnp.float32)
    y0_ref[...] = y * lax.rsqrt(_deg_of(degp_ref))


def _scale_mid_body(degp_ref, zp_ref, o_ref):
    z = zp_ref[0] + zp_ref[1]
    o_ref[...] = z / _deg_of(degp_ref)


def _scale_out_body(degp_ref, zp_ref, b_ref, o_ref):
    z = zp_ref[0] + zp_ref[1]
    o_ref[...] = z * lax.rsqrt(_deg_of(degp_ref)) + b_ref[...]


def _deg_spec():
    return pl.BlockSpec((NC, BN, NL), lambda i: (0, i, 0))


def _zp_spec():
    return pl.BlockSpec((NC, BN, DP), lambda i: (0, i, 0))


def _row_spec():
    return pl.BlockSpec((BN, DP), lambda i: (i, 0))


def _mm_scale(degp, xp, wp):
    return pl.pallas_call(
        _mm_scale_body, grid=(NP // BN,),
        in_specs=[_deg_spec(),
                  pl.BlockSpec((BN, D_IN), lambda i: (i, 0)),
                  pl.BlockSpec((DP, D_IN), lambda i: (0, 0))],
        out_specs=_row_spec(),
        out_shape=jax.ShapeDtypeStruct((NP, DP), jnp.float32),
    )(degp, xp, wp)


def _scale_mid(degp, zp):
    return pl.pallas_call(
        _scale_mid_body, grid=(NP // BN,),
        in_specs=[_deg_spec(), _zp_spec()],
        out_specs=_row_spec(),
        out_shape=jax.ShapeDtypeStruct((NP, DP), jnp.float32),
    )(degp, zp)


def _scale_out(degp, zp, bp):
    return pl.pallas_call(
        _scale_out_body, grid=(NP // BN,),
        in_specs=[_deg_spec(), _zp_spec(),
                  pl.BlockSpec((1, DP), lambda i: (0, 0))],
        out_specs=_row_spec(),
        out_shape=jax.ShapeDtypeStruct((NP, DP), jnp.float32),
    )(degp, zp, bp)


def kernel(features, edge_index, W, b):
    E = edge_index.shape[1]
    e_pad = ((E + NW * CHUNK - 1) // (NW * CHUNK)) * (NW * CHUNK)
    deg_kernel = _make_deg_kernel(e_pad)
    hop_kernel = _make_hop_kernel(e_pad)

    src = edge_index[0]
    dst = edge_index[1]
    if e_pad != E:
        pad_idx = jnp.full((e_pad - E,), N, jnp.int32)
        src = jnp.concatenate([src, pad_idx])
        dst = jnp.concatenate([dst, pad_idx])
    nchunks = _num_chunks(e_pad)
    src = src.reshape(NW, nchunks, CHUNK)
    dst = dst.reshape(NW, nchunks, CHUNK)

    xp = jnp.zeros((NP, D_IN), jnp.float32).at[:N].set(features)
    wp = jnp.zeros((DP, D_IN), jnp.float32).at[:D_OUT].set(W)
    bp = jnp.zeros((1, DP), jnp.float32).at[0, :D_OUT].set(b)
    ones_rows = jnp.ones((CHUNK, NL), jnp.float32)
    zeros_deg = jnp.zeros((ROWS_PER_TILE, NL), jnp.float32)
    zeros_hop = jnp.zeros((ROWS_PER_TILE, DP), jnp.float32)

    degp = deg_kernel(dst, ones_rows, zeros_deg)
    degp = degp.reshape(NC, NP, NL)

    y0 = _mm_scale(degp, xp, wp)
    z1 = hop_kernel(y0, src, dst, zeros_hop).reshape(NC, NP, DP)
    y1 = _scale_mid(degp, z1)
    z2 = hop_kernel(y1, src, dst, zeros_hop).reshape(NC, NP, DP)
    out = _scale_out(degp, z2, bp)
    return out[:N, :D_OUT]


# speedup vs baseline: 1.5032x; 1.0567x over previous
import functools

import jax
import jax.numpy as jnp
from jax import lax
from jax.experimental import pallas as pl
from jax.experimental.pallas import tpu as pltpu
from jax.experimental.pallas import tpu_sc as plsc

N = 10000
D_IN = 128
D_OUT = 40

NC, NS, NL = 2, 16, 16
NW = NC * NS
NP = 10240
NSP = 10048
DP = 40
CHUNK = 128
ROWS_PER_TILE = NSP // NS
BN = 2560

_mesh = plsc.VectorSubcoreMesh(core_axis_name="c", subcore_axis_name="s")
_sc_params = pltpu.CompilerParams(use_tc_tiling_on_sc=False)


def _num_chunks(e_pad):
    return e_pad // (NW * CHUNK)


def _make_deg_kernel(e_pad):
    nchunks = _num_chunks(e_pad)
    depth = 4

    @functools.partial(
        pl.kernel,
        out_type=jax.ShapeDtypeStruct((NC * NP, NL), jnp.float32),
        scratch_types=[
            pltpu.VMEM_SHARED((NSP, NL), jnp.float32),
            pltpu.VMEM((CHUNK, NL), jnp.float32),
            pltpu.VMEM((nchunks, CHUNK), jnp.int32),
            pltpu.VMEM((ROWS_PER_TILE, NL), jnp.float32),
            pltpu.SemaphoreType.DMA,
        ],
        mesh=_mesh,
        compiler_params=_sc_params,
    )
    def deg_kernel(dst_hbm, ones_hbm, zeros_hbm, out_hbm,
                   deg_sp, ones_v, dst_slab, zrows_v, sem_s):
        cid = lax.axis_index("c")
        sid = lax.axis_index("s")
        wid = sid * NC + cid
        rbase = sid * ROWS_PER_TILE
        pltpu.sync_copy(ones_hbm, ones_v)
        pltpu.sync_copy(zeros_hbm, zrows_v)
        pltpu.sync_copy(zrows_v, deg_sp.at[pl.ds(rbase, ROWS_PER_TILE)])
        pltpu.sync_copy(dst_hbm.at[wid], dst_slab)
        plsc.subcore_barrier()

        def wait_one():
            pltpu.make_async_copy(zeros_hbm.at[pl.ds(0, CHUNK)], ones_v,
                                  sem_s).wait()

        def body(j, carry):
            @pl.when(j >= depth)
            def _():
                wait_one()
            pltpu.async_copy(ones_v, deg_sp.at[dst_slab.at[j]], sem_s,
                             add=True)
            return carry

        lax.fori_loop(0, nchunks, body, 0)
        for _ in range(depth):
            wait_one()
        plsc.subcore_barrier()
        pltpu.sync_copy(deg_sp.at[pl.ds(rbase, ROWS_PER_TILE)],
                        out_hbm.at[pl.ds(cid * NP + rbase, ROWS_PER_TILE)])

    return deg_kernel


NBUF = 4


def _edge_loop(nchunks, yn_ref, acc_sp, src_slab, dst_slab, rows,
               sem_g, sem_s, dummy_hbm):
    def gather(j, slot):
        pltpu.async_copy(yn_ref.at[src_slab.at[j]], rows.at[slot], sem_g)

    def scatter(j, slot):
        pltpu.async_copy(rows.at[slot], acc_sp.at[dst_slab.at[j]], sem_s,
                         add=True)

    def wait(sem):
        pltpu.make_async_copy(dummy_hbm.at[pl.ds(0, CHUNK)], rows.at[0],
                              sem).wait()

    gather(0, 0)
    gather(1, 1)
    gather(2, 2)

    def body(j, carry):
        wait(sem_g)

        @pl.when(j >= 1)
        def _():
            wait(sem_s)

        @pl.when(j + 3 < nchunks)
        def _():
            gather(j + 3, lax.rem(j + 3, NBUF))

        scatter(j, lax.rem(j, NBUF))
        return carry

    lax.fori_loop(0, nchunks, body, 0)
    wait(sem_s)


def _make_hop_kernel(e_pad):
    nchunks = _num_chunks(e_pad)

    @functools.partial(
        pl.kernel,
        out_type=jax.ShapeDtypeStruct((NC * NP, DP), jnp.float32),
        scratch_types=[
            pltpu.VMEM_SHARED((NSP, DP), jnp.float32),
            pltpu.VMEM_SHARED((NSP, DP), jnp.float32),
            pltpu.VMEM((nchunks, CHUNK), jnp.int32),
            pltpu.VMEM((nchunks, CHUNK), jnp.int32),
            pltpu.VMEM((NBUF, CHUNK, DP), jnp.float32),
            pltpu.SemaphoreType.DMA,
            pltpu.SemaphoreType.DMA,
        ],
        mesh=_mesh,
        compiler_params=_sc_params,
    )
    def hop_kernel(yn_hbm, src_hbm, dst_hbm, zeros_hbm, out_hbm,
                   yn_sp, acc_sp, src_slab, dst_slab, rows, sem_g, sem_s):
        cid = lax.axis_index("c")
        sid = lax.axis_index("s")
        wid = sid * NC + cid
        rbase = sid * ROWS_PER_TILE
        pltpu.sync_copy(yn_hbm.at[pl.ds(rbase, ROWS_PER_TILE)],
                        yn_sp.at[pl.ds(rbase, ROWS_PER_TILE)])
        pltpu.sync_copy(zeros_hbm, acc_sp.at[pl.ds(rbase, ROWS_PER_TILE)])
        pltpu.sync_copy(src_hbm.at[wid], src_slab)
        pltpu.sync_copy(dst_hbm.at[wid], dst_slab)
        plsc.subcore_barrier()
        _edge_loop(nchunks, yn_sp, acc_sp, src_slab, dst_slab, rows,
                   sem_g, sem_s, yn_hbm)
        plsc.subcore_barrier()
        pltpu.sync_copy(acc_sp.at[pl.ds(rbase, ROWS_PER_TILE)],
                        out_hbm.at[pl.ds(cid * NP + rbase, ROWS_PER_TILE)])

    return hop_kernel


def _deg_of(degp_ref):
    deg = degp_ref[0, :, 0:1] + degp_ref[1, :, 0:1]
    return jnp.maximum(deg, 1.0)


def _mm_scale_body(degp_ref, x_ref, w_ref, y0_ref):
    y = lax.dot_general(
        x_ref[...], w_ref[...], (((1,), (1,)), ((), ())),
        preferred_element_type=jnp.float32)
    y0_ref[...] = y * lax.rsqrt(_deg_of(degp_ref))


def _scale_mid_body(degp_ref, zp_ref, o_ref):
    z = zp_ref[0] + zp_ref[1]
    o_ref[...] = z / _deg_of(degp_ref)


def _scale_out_body(degp_ref, zp_ref, b_ref, o_ref):
    z = zp_ref[0] + zp_ref[1]
    o_ref[...] = z * lax.rsqrt(_deg_of(degp_ref)) + b_ref[...]


def _deg_spec():
    return pl.BlockSpec((NC, BN, NL), lambda i: (0, i, 0))


def _zp_spec():
    return pl.BlockSpec((NC, BN, DP), lambda i: (0, i, 0))


def _row_spec():
    return pl.BlockSpec((BN, DP), lambda i: (i, 0))


def _mm_scale(degp, xp, wp):
    return pl.pallas_call(
        _mm_scale_body, grid=(NP // BN,),
        in_specs=[_deg_spec(),
                  pl.BlockSpec((BN, D_IN), lambda i: (i, 0)),
                  pl.BlockSpec((DP, D_IN), lambda i: (0, 0))],
        out_specs=_row_spec(),
        out_shape=jax.ShapeDtypeStruct((NP, DP), jnp.float32),
    )(degp, xp, wp)


def _scale_mid(degp, zp):
    return pl.pallas_call(
        _scale_mid_body, grid=(NP // BN,),
        in_specs=[_deg_spec(), _zp_spec()],
        out_specs=_row_spec(),
        out_shape=jax.ShapeDtypeStruct((NP, DP), jnp.float32),
    )(degp, zp)


def _scale_out(degp, zp, bp):
    return pl.pallas_call(
        _scale_out_body, grid=(NP // BN,),
        in_specs=[_deg_spec(), _zp_spec(),
                  pl.BlockSpec((1, DP), lambda i: (0, 0))],
        out_specs=_row_spec(),
        out_shape=jax.ShapeDtypeStruct((NP, DP), jnp.float32),
    )(degp, zp, bp)


def kernel(features, edge_index, W, b):
    E = edge_index.shape[1]
    e_pad = ((E + NW * CHUNK - 1) // (NW * CHUNK)) * (NW * CHUNK)
    deg_kernel = _make_deg_kernel(e_pad)
    hop_kernel = _make_hop_kernel(e_pad)

    src = edge_index[0]
    dst = edge_index[1]
    if e_pad != E:
        pad_idx = jnp.full((e_pad - E,), N, jnp.int32)
        src = jnp.concatenate([src, pad_idx])
        dst = jnp.concatenate([dst, pad_idx])
    nchunks = _num_chunks(e_pad)
    src = src.reshape(NW, nchunks, CHUNK)
    dst = dst.reshape(NW, nchunks, CHUNK)

    xp = jnp.zeros((NP, D_IN), jnp.float32).at[:N].set(features)
    wp = jnp.zeros((DP, D_IN), jnp.float32).at[:D_OUT].set(W)
    bp = jnp.zeros((1, DP), jnp.float32).at[0, :D_OUT].set(b)
    ones_rows = jnp.ones((CHUNK, NL), jnp.float32)
    zeros_deg = jnp.zeros((ROWS_PER_TILE, NL), jnp.float32)
    zeros_hop = jnp.zeros((ROWS_PER_TILE, DP), jnp.float32)

    degp = deg_kernel(dst, ones_rows, zeros_deg)
    degp = degp.reshape(NC, NP, NL)

    y0 = _mm_scale(degp, xp, wp)
    z1 = hop_kernel(y0, src, dst, zeros_hop).reshape(NC, NP, DP)
    y1 = _scale_mid(degp, z1)
    z2 = hop_kernel(y1, src, dst, zeros_hop).reshape(NC, NP, DP)
    out = _scale_out(degp, z2, bp)
    return out[:N, :D_OUT]


# speedup vs baseline: 1.5921x; 1.0591x over previous
import functools

import jax
import jax.numpy as jnp
from jax import lax
from jax.experimental import pallas as pl
from jax.experimental.pallas import tpu as pltpu
from jax.experimental.pallas import tpu_sc as plsc

N = 10000
D_IN = 128
D_OUT = 40

NC, NS, NL = 2, 16, 16
NW = NC * NS
NP = 10240
NSP = 10048
DP = 40
CHUNK = 128
ROWS_PER_TILE = NSP // NS
BN = 2560

_mesh = plsc.VectorSubcoreMesh(core_axis_name="c", subcore_axis_name="s")
_sc_params = pltpu.CompilerParams(use_tc_tiling_on_sc=False)


def _num_chunks(e_pad):
    return e_pad // (NW * CHUNK)


def _make_deg_kernel(e_pad):
    nchunks = _num_chunks(e_pad)
    depth = 4

    @functools.partial(
        pl.kernel,
        out_type=jax.ShapeDtypeStruct((NC * NP, NL), jnp.float32),
        scratch_types=[
            pltpu.VMEM_SHARED((NSP, NL), jnp.float32),
            pltpu.VMEM((CHUNK, NL), jnp.float32),
            pltpu.VMEM((nchunks, CHUNK), jnp.int32),
            pltpu.VMEM((ROWS_PER_TILE, NL), jnp.float32),
            pltpu.SemaphoreType.DMA,
        ],
        mesh=_mesh,
        compiler_params=_sc_params,
    )
    def deg_kernel(dst_hbm, ones_hbm, zeros_hbm, out_hbm,
                   deg_sp, ones_v, dst_slab, zrows_v, sem_s):
        cid = lax.axis_index("c")
        sid = lax.axis_index("s")
        wid = sid * NC + cid
        rbase = sid * ROWS_PER_TILE
        pltpu.sync_copy(ones_hbm, ones_v)
        pltpu.sync_copy(zeros_hbm, zrows_v)
        pltpu.sync_copy(zrows_v, deg_sp.at[pl.ds(rbase, ROWS_PER_TILE)])
        pltpu.sync_copy(dst_hbm.at[wid], dst_slab)
        plsc.subcore_barrier()

        def wait_one():
            pltpu.make_async_copy(zeros_hbm.at[pl.ds(0, CHUNK)], ones_v,
                                  sem_s).wait()

        def body(j, carry):
            @pl.when(j >= depth)
            def _():
                wait_one()
            pltpu.async_copy(ones_v, deg_sp.at[dst_slab.at[j]], sem_s,
                             add=True)
            return carry

        lax.fori_loop(0, nchunks, body, 0)
        for _ in range(depth):
            wait_one()
        plsc.subcore_barrier()
        pltpu.sync_copy(deg_sp.at[pl.ds(rbase, ROWS_PER_TILE)],
                        out_hbm.at[pl.ds(cid * NP + rbase, ROWS_PER_TILE)])

    return deg_kernel


NBUF = 8


def _edge_loop(nchunks, yn_ref, acc_sp, src_slab, dst_slab, rows,
               sem_g, sem_s, dummy_hbm):
    def gather(j, slot):
        pltpu.async_copy(yn_ref.at[src_slab.at[j]], rows.at[slot], sem_g)

    def scatter(j, slot):
        pltpu.async_copy(rows.at[slot], acc_sp.at[dst_slab.at[j]], sem_s,
                         add=True)

    def wait(sem):
        pltpu.make_async_copy(dummy_hbm.at[pl.ds(0, CHUNK)], rows.at[0],
                              sem).wait()

    for p in range(5):
        gather(p, p)

    def body(j, carry):
        wait(sem_g)
        scatter(j, lax.rem(j, NBUF))

        @pl.when(j >= 2)
        def _():
            wait(sem_s)

        @pl.when(j + 5 < nchunks)
        def _():
            gather(j + 5, lax.rem(j + 5, NBUF))

        return carry

    lax.fori_loop(0, nchunks, body, 0)
    wait(sem_s)
    wait(sem_s)


def _make_hop_kernel(e_pad):
    nchunks = _num_chunks(e_pad)

    @functools.partial(
        pl.kernel,
        out_type=jax.ShapeDtypeStruct((NC * NP, DP), jnp.float32),
        scratch_types=[
            pltpu.VMEM_SHARED((NSP, DP), jnp.float32),
            pltpu.VMEM_SHARED((NSP, DP), jnp.float32),
            pltpu.VMEM((nchunks, CHUNK), jnp.int32),
            pltpu.VMEM((nchunks, CHUNK), jnp.int32),
            pltpu.VMEM((NBUF, CHUNK, DP), jnp.float32),
            pltpu.SemaphoreType.DMA,
            pltpu.SemaphoreType.DMA,
        ],
        mesh=_mesh,
        compiler_params=_sc_params,
    )
    def hop_kernel(yn_hbm, src_hbm, dst_hbm, zeros_hbm, out_hbm,
                   yn_sp, acc_sp, src_slab, dst_slab, rows, sem_g, sem_s):
        cid = lax.axis_index("c")
        sid = lax.axis_index("s")
        wid = sid * NC + cid
        rbase = sid * ROWS_PER_TILE
        pltpu.sync_copy(yn_hbm.at[pl.ds(rbase, ROWS_PER_TILE)],
                        yn_sp.at[pl.ds(rbase, ROWS_PER_TILE)])
        pltpu.sync_copy(zeros_hbm, acc_sp.at[pl.ds(rbase, ROWS_PER_TILE)])
        pltpu.sync_copy(src_hbm.at[wid], src_slab)
        pltpu.sync_copy(dst_hbm.at[wid], dst_slab)
        plsc.subcore_barrier()
        _edge_loop(nchunks, yn_sp, acc_sp, src_slab, dst_slab, rows,
                   sem_g, sem_s, yn_hbm)
        plsc.subcore_barrier()
        pltpu.sync_copy(acc_sp.at[pl.ds(rbase, ROWS_PER_TILE)],
                        out_hbm.at[pl.ds(cid * NP + rbase, ROWS_PER_TILE)])

    return hop_kernel


def _deg_of(degp_ref):
    deg = degp_ref[0, :, 0:1] + degp_ref[1, :, 0:1]
    return jnp.maximum(deg, 1.0)


def _mm_scale_body(degp_ref, x_ref, w_ref, y0_ref):
    y = lax.dot_general(
        x_ref[...], w_ref[...], (((1,), (1,)), ((), ())),
        preferred_element_type=jnp.float32)
    y0_ref[...] = y * lax.rsqrt(_deg_of(degp_ref))


def _scale_mid_body(degp_ref, zp_ref, o_ref):
    z = zp_ref[0] + zp_ref[1]
    o_ref[...] = z / _deg_of(degp_ref)


def _scale_out_body(degp_ref, zp_ref, b_ref, o_ref):
    z = zp_ref[0] + zp_ref[1]
    o_ref[...] = z * lax.rsqrt(_deg_of(degp_ref)) + b_ref[...]


def _deg_spec():
    return pl.BlockSpec((NC, BN, NL), lambda i: (0, i, 0))


def _zp_spec():
    return pl.BlockSpec((NC, BN, DP), lambda i: (0, i, 0))


def _row_spec():
    return pl.BlockSpec((BN, DP), lambda i: (i, 0))


def _mm_scale(degp, xp, wp):
    return pl.pallas_call(
        _mm_scale_body, grid=(NP // BN,),
        in_specs=[_deg_spec(),
                  pl.BlockSpec((BN, D_IN), lambda i: (i, 0)),
                  pl.BlockSpec((DP, D_IN), lambda i: (0, 0))],
        out_specs=_row_spec(),
        out_shape=jax.ShapeDtypeStruct((NP, DP), jnp.float32),
    )(degp, xp, wp)


def _scale_mid(degp, zp):
    return pl.pallas_call(
        _scale_mid_body, grid=(NP // BN,),
        in_specs=[_deg_spec(), _zp_spec()],
        out_specs=_row_spec(),
        out_shape=jax.ShapeDtypeStruct((NP, DP), jnp.float32),
    )(degp, zp)


def _scale_out(degp, zp, bp):
    return pl.pallas_call(
        _scale_out_body, grid=(NP // BN,),
        in_specs=[_deg_spec(), _zp_spec(),
                  pl.BlockSpec((1, DP), lambda i: (0, 0))],
        out_specs=_row_spec(),
        out_shape=jax.ShapeDtypeStruct((NP, DP), jnp.float32),
    )(degp, zp, bp)


def kernel(features, edge_index, W, b):
    E = edge_index.shape[1]
    e_pad = ((E + NW * CHUNK - 1) // (NW * CHUNK)) * (NW * CHUNK)
    deg_kernel = _make_deg_kernel(e_pad)
    hop_kernel = _make_hop_kernel(e_pad)

    src = edge_index[0]
    dst = edge_index[1]
    if e_pad != E:
        pad_idx = jnp.full((e_pad - E,), N, jnp.int32)
        src = jnp.concatenate([src, pad_idx])
        dst = jnp.concatenate([dst, pad_idx])
    nchunks = _num_chunks(e_pad)
    src = src.reshape(NW, nchunks, CHUNK)
    dst = dst.reshape(NW, nchunks, CHUNK)

    xp = jnp.zeros((NP, D_IN), jnp.float32).at[:N].set(features)
    wp = jnp.zeros((DP, D_IN), jnp.float32).at[:D_OUT].set(W)
    bp = jnp.zeros((1, DP), jnp.float32).at[0, :D_OUT].set(b)
    ones_rows = jnp.ones((CHUNK, NL), jnp.float32)
    zeros_deg = jnp.zeros((ROWS_PER_TILE, NL), jnp.float32)
    zeros_hop = jnp.zeros((ROWS_PER_TILE, DP), jnp.float32)

    degp = deg_kernel(dst, ones_rows, zeros_deg)
    degp = degp.reshape(NC, NP, NL)

    y0 = _mm_scale(degp, xp, wp)
    z1 = hop_kernel(y0, src, dst, zeros_hop).reshape(NC, NP, DP)
    y1 = _scale_mid(degp, z1)
    z2 = hop_kernel(y1, src, dst, zeros_hop).reshape(NC, NP, DP)
    out = _scale_out(degp, z2, bp)
    return out[:N, :D_OUT]
